# baseline (speedup 1.0000x reference)
import jax
import jax.numpy as jnp
from jax import lax
from jax.experimental import pallas as pl
from jax.experimental.pallas import tpu as pltpu
from jax.experimental.pallas import tpu_sc as plsc

N = 100000
B = 128
EPS = 1e-5
SLOPE = 0.01

NC = 2
NS = 16
LANES = 16

NPAD = 100352
STRIPE = NPAD // NS
ZCH = 392

GROUP = 512
SUB = 128
GSUB = GROUP // SUB
PAIR = 2 * GROUP
PSUB = 2 * GSUB

GP_SPMM = 196
EPAD = NS * GP_SPMM * GROUP
GP_DEG = EPAD // (NC * NS * GROUP)

RB = 2000
NRB = N // RB


_MESH = plsc.VectorSubcoreMesh(core_axis_name="c", subcore_axis_name="s")
_SC_PARAMS = pltpu.CompilerParams(use_tc_tiling_on_sc=False)


def _deg_body(col2, degp, acc, zb, colp, ones_v, ssem0, ssem1):
    c = lax.axis_index("c")
    s = lax.axis_index("s")

    @pl.loop(0, ZCH)
    def _(i):
        zb[i] = jnp.zeros((LANES,), jnp.float32)

    @pl.loop(0, SUB)
    def _(i):
        ones_v[i] = jnp.ones((LANES,), jnp.float32)

    zcps = [
        pltpu.async_copy(
            zb,
            acc.at[pl.ds(pl.multiple_of(s * STRIPE + k * ZCH, ZCH), ZCH)],
            ssem0)
        for k in range(STRIPE // ZCH)
    ]
    for cp in zcps:
        cp.wait()
    plsc.subcore_barrier()

    w = s * NC + c
    base = w * GP_DEG * GSUB

    @pl.loop(0, GP_DEG // 2)
    def _(g):
        pltpu.sync_copy(
            col2.at[pl.ds(pl.multiple_of(base + g * PSUB, PSUB), PSUB)], colp)
        scps0 = [pltpu.async_copy(ones_v, acc.at[colp.at[u]], ssem0, add=True)
                 for u in range(GSUB)]
        scps1 = [pltpu.async_copy(ones_v, acc.at[colp.at[GSUB + u]], ssem1,
                                  add=True)
                 for u in range(GSUB)]
        for cp in scps0:
            cp.wait()
        for cp in scps1:
            cp.wait()

    plsc.subcore_barrier()
    r = pl.ds(pl.multiple_of(s * STRIPE, STRIPE), STRIPE)
    pltpu.sync_copy(acc.at[r], degp.at[c, r])


_deg_call = pl.kernel(
    _deg_body,
    out_type=jax.ShapeDtypeStruct((NC, NPAD, LANES), jnp.float32),
    mesh=_MESH,
    compiler_params=_SC_PARAMS,
    scratch_types=[
        pltpu.VMEM_SHARED((NPAD, LANES), jnp.float32),
        pltpu.VMEM((ZCH, LANES), jnp.float32),
        pltpu.VMEM((PSUB, SUB), jnp.int32),
        pltpu.VMEM((SUB, LANES), jnp.float32),
        pltpu.SemaphoreType.DMA,
        pltpu.SemaphoreType.DMA,
    ],
)


def _make_spmm(nblk):
    bpc = nblk // NC

    def body(*refs):
        ys = refs[:nblk]
        row_h, col2, z = refs[nblk:nblk + 3]
        (acc, zb, rowp, colp, gbuf0, gbuf1,
         isem, gsem0, gsem1, ssem0, ssem1, wsem) = refs[nblk + 3:]
        c = lax.axis_index("c")
        s = lax.axis_index("s")

        @pl.loop(0, ZCH)
        def _(i):
            zb[i] = jnp.zeros((LANES,), jnp.float32)

        for core in range(NC):
            @pl.when(c == core)
            def _():
                for bi in range(bpc):
                    blk = bi * NC + core
                    ytab = ys[blk]

                    zcps = [
                        pltpu.async_copy(
                            zb,
                            acc.at[pl.ds(
                                pl.multiple_of(s * STRIPE + k * ZCH, ZCH),
                                ZCH)],
                            wsem)
                        for k in range(STRIPE // ZCH)
                    ]
                    for cp in zcps:
                        cp.wait()
                    plsc.subcore_barrier()

                    @pl.loop(0, GP_SPMM // 2)
                    def _(i):
                        eb = pl.multiple_of(
                            s * GP_SPMM * GROUP + i * PAIR, PAIR)
                        eb128 = pl.multiple_of(
                            s * GP_SPMM * GSUB + i * PSUB, PSUB)
                        icps = [pltpu.async_copy(row_h.at[pl.ds(eb, PAIR)],
                                                 rowp, isem),
                                pltpu.async_copy(col2.at[pl.ds(eb128, PSUB)],
                                                 colp, isem)]
                        for cp in icps:
                            cp.wait()

                        gcps0 = [pltpu.async_copy(
                            ytab.at[rowp.at[pl.ds(u * SUB, SUB)]],
                            gbuf0.at[pl.ds(u * SUB, SUB)], gsem0)
                            for u in range(GSUB)]
                        gcps1 = [pltpu.async_copy(
                            ytab.at[rowp.at[pl.ds(GROUP + u * SUB, SUB)]],
                            gbuf1.at[pl.ds(u * SUB, SUB)], gsem1)
                            for u in range(GSUB)]
                        for cp in gcps0:
                            cp.wait()
                        scps0 = [pltpu.async_copy(
                            gbuf0.at[pl.ds(u * SUB, SUB)],
                            acc.at[colp.at[u]], ssem0, add=True)
                            for u in range(GSUB)]
                        for cp in gcps1:
                            cp.wait()
                        scps1 = [pltpu.async_copy(
                            gbuf1.at[pl.ds(u * SUB, SUB)],
                            acc.at[colp.at[GSUB + u]], ssem1, add=True)
                            for u in range(GSUB)]
                        for cp in scps0:
                            cp.wait()
                        for cp in scps1:
                            cp.wait()

                    plsc.subcore_barrier()

                    r = pl.ds(pl.multiple_of(s * STRIPE, STRIPE), STRIPE)
                    pltpu.sync_copy(acc.at[r], z.at[blk, r])
                    plsc.subcore_barrier()

    return pl.kernel(
        body,
        out_type=jax.ShapeDtypeStruct((nblk, NPAD, LANES), jnp.float32),
        mesh=_MESH,
        compiler_params=_SC_PARAMS,
        scratch_types=[
            pltpu.VMEM_SHARED((NPAD, LANES), jnp.float32),
            pltpu.VMEM((ZCH, LANES), jnp.float32),
            pltpu.VMEM((PAIR,), jnp.int32),
            pltpu.VMEM((PSUB, SUB), jnp.int32),
            pltpu.VMEM((GROUP, LANES), jnp.float32),
            pltpu.VMEM((GROUP, LANES), jnp.float32),
            pltpu.SemaphoreType.DMA,
            pltpu.SemaphoreType.DMA,
            pltpu.SemaphoreType.DMA,
            pltpu.SemaphoreType.DMA,
            pltpu.SemaphoreType.DMA,
            pltpu.SemaphoreType.DMA,
        ],
    )


_spmm64 = _make_spmm(4)
_spmm32 = _make_spmm(2)


def _dinv(dp_ref):
    return lax.rsqrt(dp_ref[0] + dp_ref[1] + 1.0)


def _make_t1_body(nblk):
    def _t1_body(x_ref, w_ref, dp_ref, *y_refs):
        dinv = _dinv(dp_ref)
        y = jnp.dot(x_ref[...], w_ref[...],
                    preferred_element_type=jnp.float32)
        for j in range(nblk):
            y_refs[j][...] = y[:, j * LANES:(j + 1) * LANES] * dinv

    return _t1_body


def _t1_call(din, dout):
    nblk = dout // LANES
    return pl.pallas_call(
        _make_t1_body(nblk),
        grid=(NRB,),
        in_specs=[
            pl.BlockSpec((RB, din), lambda i: (i, 0)),
            pl.BlockSpec((din, dout), lambda i: (0, 0)),
            pl.BlockSpec((NC, RB, LANES), lambda i: (0, i, 0)),
        ],
        out_specs=[pl.BlockSpec((RB, LANES), lambda i: (i, 0))] * nblk,
        out_shape=[jax.ShapeDtypeStruct((N, LANES), jnp.float32)] * nblk,
    )


def _make_t2_body(nblk):
    def _t2_body(*refs):
        z_ref = refs[0]
        y_refs = refs[1:1 + nblk]
        dp_ref, b_ref = refs[1 + nblk:3 + nblk]
        o_refs = refs[3 + nblk:3 + 2 * nblk]
        st_ref = refs[3 + 2 * nblk]
        i = pl.program_id(0)
        dinv = _dinv(dp_ref)

        @pl.when(i == 0)
        def _():
            st_ref[...] = jnp.zeros_like(st_ref)

        for j in range(nblk):
            sl = pl.ds(j * LANES, LANES)
            o = (z_ref[j] + y_refs[j][...]) * dinv + b_ref[:, sl]
            o_refs[j][...] = o
            st_ref[0:1, sl] += jnp.sum(o, axis=0, keepdims=True)
            st_ref[1:2, sl] += jnp.sum(o * o, axis=0, keepdims=True)

    return _t2_body


def _t2_call(d):
    nblk = d // LANES
    return pl.pallas_call(
        _make_t2_body(nblk),
        grid=(NRB,),
        in_specs=[
            pl.BlockSpec((nblk, RB, LANES), lambda i: (0, i, 0)),
            *([pl.BlockSpec((RB, LANES), lambda i: (i, 0))] * nblk),
            pl.BlockSpec((NC, RB, LANES), lambda i: (0, i, 0)),
            pl.BlockSpec((1, d), lambda i: (0, 0)),
        ],
        out_specs=[
            *([pl.BlockSpec((RB, LANES), lambda i: (i, 0))] * nblk),
            pl.BlockSpec((8, d), lambda i: (0, 0)),
        ],
        out_shape=[
            *([jax.ShapeDtypeStruct((N, LANES), jnp.float32)] * nblk),
            jax.ShapeDtypeStruct((8, d), jnp.float32),
        ],
    )


def _bn_act_block(o, st_ref, g_ref, be_ref, sl):
    mean = st_ref[0:1, sl] * (1.0 / N)
    var = st_ref[1:2, sl] * (1.0 / N) - mean * mean
    xn = (o - mean) * lax.rsqrt(var + EPS) * g_ref[:, sl] + be_ref[:, sl]
    return jnp.where(xn >= 0, xn, SLOPE * xn)


def _make_t3_body(nin, nout):
    def _t3_body(*refs):
        o_refs = refs[:nin]
        st_ref, g_ref, be_ref, w_ref, dp_ref = refs[nin:nin + 5]
        y_refs = refs[nin + 5:]
        dinv = _dinv(dp_ref)
        hs = [_bn_act_block(o_refs[j][...], st_ref, g_ref, be_ref,
                            pl.ds(j * LANES, LANES))
              for j in range(nin)]
        h = jnp.concatenate(hs, axis=1)
        y = jnp.dot(h, w_ref[...], preferred_element_type=jnp.float32)
        for k in range(nout):
            y_refs[k][...] = y[:, k * LANES:(k + 1) * LANES] * dinv

    return _t3_body


def _t3_call(din, dout):
    nin = din // LANES
    nout = dout // LANES
    return pl.pallas_call(
        _make_t3_body(nin, nout),
        grid=(NRB,),
        in_specs=[
            *([pl.BlockSpec((RB, LANES), lambda i: (i, 0))] * nin),
            pl.BlockSpec((8, din), lambda i: (0, 0)),
            pl.BlockSpec((1, din), lambda i: (0, 0)),
            pl.BlockSpec((1, din), lambda i: (0, 0)),
            pl.BlockSpec((din, dout), lambda i: (0, 0)),
            pl.BlockSpec((NC, RB, LANES), lambda i: (0, i, 0)),
        ],
        out_specs=[pl.BlockSpec((RB, LANES), lambda i: (i, 0))] * nout,
        out_shape=[jax.ShapeDtypeStruct((N, LANES), jnp.float32)] * nout,
    )


def _make_t4_body(nblk):
    def _t4_body(*refs):
        o_refs = refs[:nblk]
        st_ref, g_ref, be_ref, bt_ref = refs[nblk:nblk + 4]
        out_ref, acc, cnt = refs[nblk + 4:]
        i = pl.program_id(0)

        @pl.when(i == 0)
        def _():
            acc[...] = jnp.zeros_like(acc)
            cnt[...] = jnp.zeros_like(cnt)

        hs = [_bn_act_block(o_refs[j][...], st_ref, g_ref, be_ref,
                            pl.ds(j * LANES, LANES))
              for j in range(nblk)]
        h = jnp.concatenate(hs, axis=1)
        onehot = (bt_ref[...] == lax.broadcasted_iota(jnp.int32, (RB, B), 1)
                  ).astype(jnp.float32)
        acc[...] += lax.dot_general(onehot, h, (((0,), (0,)), ((), ())),
                                    preferred_element_type=jnp.float32)
        cnt[...] += lax.dot_general(onehot, jnp.ones((RB, 1), jnp.float32),
                                    (((0,), (0,)), ((), ())),
                                    preferred_element_type=jnp.float32)

        @pl.when(i == NRB - 1)
        def _():
            out_ref[...] = acc[...] / jnp.maximum(cnt[...], 1.0)

    return _t4_body


def _t4_call(d):
    nblk = d // LANES
    return pl.pallas_call(
        _make_t4_body(nblk),
        grid=(NRB,),
        in_specs=[
            *([pl.BlockSpec((RB, LANES), lambda i: (i, 0))] * nblk),
            pl.BlockSpec((8, d), lambda i: (0, 0)),
            pl.BlockSpec((1, d), lambda i: (0, 0)),
            pl.BlockSpec((1, d), lambda i: (0, 0)),
            pl.BlockSpec((RB, 1), lambda i: (i, 0)),
        ],
        out_specs=pl.BlockSpec((B, d), lambda i: (0, 0)),
        out_shape=jax.ShapeDtypeStruct((B, d), jnp.float32),
        scratch_shapes=[
            pltpu.VMEM((B, d), jnp.float32),
            pltpu.VMEM((B, 1), jnp.float32),
        ],
    )


def kernel(x, edge_index, batch,
           W1, b1, g1, be1, W2, b2, g2, be2, W3, b3, g3, be3):
    row = edge_index[0].astype(jnp.int32)
    col = edge_index[1].astype(jnp.int32)
    e = row.shape[0]
    row_p = jnp.concatenate([row, jnp.zeros((EPAD - e,), jnp.int32)])
    col_p = jnp.concatenate([col, jnp.full((EPAD - e,), NPAD - 1, jnp.int32)])
    col2 = col_p.reshape(EPAD // SUB, SUB)
    batch_c = batch.astype(jnp.int32).reshape(N, 1)

    x8 = jnp.pad(x, ((0, 0), (0, 8 - x.shape[1])))
    W18 = jnp.pad(W1, ((0, 8 - W1.shape[0]), (0, 0)))

    dp = _deg_call(col2)

    hid = W1.shape[1]
    emb = W3.shape[1]

    y1 = _t1_call(8, hid)(x8, W18, dp)
    z1 = _spmm64(*y1, row_p, col2)
    *o1, st1 = _t2_call(hid)(z1, *y1, dp, b1.reshape(1, hid))
    y2 = _t3_call(hid, hid)(*o1, st1, g1.reshape(1, hid),
                            be1.reshape(1, hid), W2, dp)
    z2 = _spmm64(*y2, row_p, col2)
    *o2, st2 = _t2_call(hid)(z2, *y2, dp, b2.reshape(1, hid))
    y3 = _t3_call(hid, emb)(*o2, st2, g2.reshape(1, hid),
                            be2.reshape(1, hid), W3, dp)
    z3 = _spmm32(*y3, row_p, col2)
    *o3, st3 = _t2_call(emb)(z3, *y3, dp, b3.reshape(1, emb))
    out = _t4_call(emb)(*o3, st3, g3.reshape(1, emb), be3.reshape(1, emb),
                        batch_c)
    return out


# speedup vs baseline: 1.0017x; 1.0017x over previous
import jax
import jax.numpy as jnp
from jax import lax
from jax.experimental import pallas as pl
from jax.experimental.pallas import tpu as pltpu
from jax.experimental.pallas import tpu_sc as plsc

N = 100000
B = 128
EPS = 1e-5
SLOPE = 0.01

NC = 2
NS = 16
LANES = 16

NPAD = 100352
STRIPE = NPAD // NS
ZCH = 392

GROUP = 512
SUB = 128
GSUB = GROUP // SUB
PAIR = 2 * GROUP
PSUB = 2 * GSUB

GP_SPMM = 196
EPAD = NS * GP_SPMM * GROUP
GP_DEG = EPAD // (NC * NS * GROUP)

RB = 2000
NRB = N // RB


_MESH = plsc.VectorSubcoreMesh(core_axis_name="c", subcore_axis_name="s")
_SC_PARAMS = pltpu.CompilerParams(use_tc_tiling_on_sc=False)


def _deg_body(col2, degp, acc, zb, colp, ones_v, ssem0, ssem1):
    c = lax.axis_index("c")
    s = lax.axis_index("s")

    @pl.loop(0, ZCH)
    def _(i):
        zb[i] = jnp.zeros((LANES,), jnp.float32)

    @pl.loop(0, GROUP)
    def _(i):
        ones_v[i] = jnp.ones((LANES,), jnp.float32)

    zcps = [
        pltpu.async_copy(
            zb,
            acc.at[pl.ds(pl.multiple_of(s * STRIPE + k * ZCH, ZCH), ZCH)],
            ssem0)
        for k in range(STRIPE // ZCH)
    ]
    for cp in zcps:
        cp.wait()
    plsc.subcore_barrier()

    w = s * NC + c
    base = w * GP_DEG * GROUP

    @pl.loop(0, GP_DEG // 2)
    def _(g):
        pltpu.sync_copy(
            col2.at[pl.ds(pl.multiple_of(base + g * PAIR, PAIR), PAIR)], colp)
        scp0 = pltpu.async_copy(ones_v, acc.at[colp.at[pl.ds(0, GROUP)]],
                                ssem0, add=True)
        scp1 = pltpu.async_copy(ones_v, acc.at[colp.at[pl.ds(GROUP, GROUP)]],
                                ssem1, add=True)
        scp0.wait()
        scp1.wait()

    plsc.subcore_barrier()
    r = pl.ds(pl.multiple_of(s * STRIPE, STRIPE), STRIPE)
    pltpu.sync_copy(acc.at[r], degp.at[c, r])


_deg_call = pl.kernel(
    _deg_body,
    out_type=jax.ShapeDtypeStruct((NC, NPAD, LANES), jnp.float32),
    mesh=_MESH,
    compiler_params=_SC_PARAMS,
    scratch_types=[
        pltpu.VMEM_SHARED((NPAD, LANES), jnp.float32),
        pltpu.VMEM((ZCH, LANES), jnp.float32),
        pltpu.VMEM((PAIR,), jnp.int32),
        pltpu.VMEM((GROUP, LANES), jnp.float32),
        pltpu.SemaphoreType.DMA,
        pltpu.SemaphoreType.DMA,
    ],
)


def _make_spmm(nblk):
    bpc = nblk // NC

    def body(*refs):
        ys = refs[:nblk]
        row2, col2, z = refs[nblk:nblk + 3]
        (acc, zb, rowp, colp, gbuf0, gbuf1,
         isem, gsem0, gsem1, ssem0, ssem1, wsem) = refs[nblk + 3:]
        c = lax.axis_index("c")
        s = lax.axis_index("s")

        @pl.loop(0, ZCH)
        def _(i):
            zb[i] = jnp.zeros((LANES,), jnp.float32)

        for core in range(NC):
            @pl.when(c == core)
            def _():
                for bi in range(bpc):
                    blk = bi * NC + core
                    ytab = ys[blk]

                    zcps = [
                        pltpu.async_copy(
                            zb,
                            acc.at[pl.ds(
                                pl.multiple_of(s * STRIPE + k * ZCH, ZCH),
                                ZCH)],
                            wsem)
                        for k in range(STRIPE // ZCH)
                    ]
                    for cp in zcps:
                        cp.wait()
                    plsc.subcore_barrier()

                    @pl.loop(0, GP_SPMM // 2)
                    def _(i):
                        eb = pl.multiple_of(
                            s * GP_SPMM * GROUP + i * PAIR, PAIR)
                        icps = [pltpu.async_copy(row2.at[pl.ds(eb, PAIR)],
                                                 rowp, isem),
                                pltpu.async_copy(col2.at[pl.ds(eb, PAIR)],
                                                 colp, isem)]
                        for cp in icps:
                            cp.wait()

                        gcp0 = pltpu.async_copy(
                            ytab.at[rowp.at[pl.ds(0, GROUP)]], gbuf0, gsem0)
                        gcp1 = pltpu.async_copy(
                            ytab.at[rowp.at[pl.ds(GROUP, GROUP)]], gbuf1,
                            gsem1)
                        gcp0.wait()
                        scp0 = pltpu.async_copy(
                            gbuf0, acc.at[colp.at[pl.ds(0, GROUP)]], ssem0,
                            add=True)
                        gcp1.wait()
                        scp1 = pltpu.async_copy(
                            gbuf1, acc.at[colp.at[pl.ds(GROUP, GROUP)]],
                            ssem1, add=True)
                        scp0.wait()
                        scp1.wait()

                    plsc.subcore_barrier()

                    r = pl.ds(pl.multiple_of(s * STRIPE, STRIPE), STRIPE)
                    pltpu.sync_copy(acc.at[r], z.at[blk, r])
                    plsc.subcore_barrier()

    return pl.kernel(
        body,
        out_type=jax.ShapeDtypeStruct((nblk, NPAD, LANES), jnp.float32),
        mesh=_MESH,
        compiler_params=_SC_PARAMS,
        scratch_types=[
            pltpu.VMEM_SHARED((NPAD, LANES), jnp.float32),
            pltpu.VMEM((ZCH, LANES), jnp.float32),
            pltpu.VMEM((PAIR,), jnp.int32),
            pltpu.VMEM((PAIR,), jnp.int32),
            pltpu.VMEM((GROUP, LANES), jnp.float32),
            pltpu.VMEM((GROUP, LANES), jnp.float32),
            pltpu.SemaphoreType.DMA,
            pltpu.SemaphoreType.DMA,
            pltpu.SemaphoreType.DMA,
            pltpu.SemaphoreType.DMA,
            pltpu.SemaphoreType.DMA,
            pltpu.SemaphoreType.DMA,
        ],
    )


_spmm64 = _make_spmm(4)
_spmm32 = _make_spmm(2)


def _dinv(dp_ref):
    return lax.rsqrt(dp_ref[0] + dp_ref[1] + 1.0)


def _make_t1_body(nblk):
    def _t1_body(x_ref, w_ref, dp_ref, *y_refs):
        dinv = _dinv(dp_ref)
        y = jnp.dot(x_ref[...], w_ref[...],
                    preferred_element_type=jnp.float32)
        for j in range(nblk):
            y_refs[j][...] = y[:, j * LANES:(j + 1) * LANES] * dinv

    return _t1_body


def _t1_call(din, dout):
    nblk = dout // LANES
    return pl.pallas_call(
        _make_t1_body(nblk),
        grid=(NRB,),
        in_specs=[
            pl.BlockSpec((RB, din), lambda i: (i, 0)),
            pl.BlockSpec((din, dout), lambda i: (0, 0)),
            pl.BlockSpec((NC, RB, LANES), lambda i: (0, i, 0)),
        ],
        out_specs=[pl.BlockSpec((RB, LANES), lambda i: (i, 0))] * nblk,
        out_shape=[jax.ShapeDtypeStruct((N, LANES), jnp.float32)] * nblk,
    )


def _make_t2_body(nblk):
    def _t2_body(*refs):
        z_ref = refs[0]
        y_refs = refs[1:1 + nblk]
        dp_ref, b_ref = refs[1 + nblk:3 + nblk]
        o_refs = refs[3 + nblk:3 + 2 * nblk]
        st_ref = refs[3 + 2 * nblk]
        i = pl.program_id(0)
        dinv = _dinv(dp_ref)

        @pl.when(i == 0)
        def _():
            st_ref[...] = jnp.zeros_like(st_ref)

        for j in range(nblk):
            sl = pl.ds(j * LANES, LANES)
            o = (z_ref[j] + y_refs[j][...]) * dinv + b_ref[:, sl]
            o_refs[j][...] = o
            st_ref[0:1, sl] += jnp.sum(o, axis=0, keepdims=True)
            st_ref[1:2, sl] += jnp.sum(o * o, axis=0, keepdims=True)

    return _t2_body


def _t2_call(d):
    nblk = d // LANES
    return pl.pallas_call(
        _make_t2_body(nblk),
        grid=(NRB,),
        in_specs=[
            pl.BlockSpec((nblk, RB, LANES), lambda i: (0, i, 0)),
            *([pl.BlockSpec((RB, LANES), lambda i: (i, 0))] * nblk),
            pl.BlockSpec((NC, RB, LANES), lambda i: (0, i, 0)),
            pl.BlockSpec((1, d), lambda i: (0, 0)),
        ],
        out_specs=[
            *([pl.BlockSpec((RB, LANES), lambda i: (i, 0))] * nblk),
            pl.BlockSpec((8, d), lambda i: (0, 0)),
        ],
        out_shape=[
            *([jax.ShapeDtypeStruct((N, LANES), jnp.float32)] * nblk),
            jax.ShapeDtypeStruct((8, d), jnp.float32),
        ],
    )


def _bn_act_block(o, st_ref, g_ref, be_ref, sl):
    mean = st_ref[0:1, sl] * (1.0 / N)
    var = st_ref[1:2, sl] * (1.0 / N) - mean * mean
    xn = (o - mean) * lax.rsqrt(var + EPS) * g_ref[:, sl] + be_ref[:, sl]
    return jnp.where(xn >= 0, xn, SLOPE * xn)


def _make_t3_body(nin, nout):
    def _t3_body(*refs):
        o_refs = refs[:nin]
        st_ref, g_ref, be_ref, w_ref, dp_ref = refs[nin:nin + 5]
        y_refs = refs[nin + 5:]
        dinv = _dinv(dp_ref)
        hs = [_bn_act_block(o_refs[j][...], st_ref, g_ref, be_ref,
                            pl.ds(j * LANES, LANES))
              for j in range(nin)]
        h = jnp.concatenate(hs, axis=1)
        y = jnp.dot(h, w_ref[...], preferred_element_type=jnp.float32)
        for k in range(nout):
            y_refs[k][...] = y[:, k * LANES:(k + 1) * LANES] * dinv

    return _t3_body


def _t3_call(din, dout):
    nin = din // LANES
    nout = dout // LANES
    return pl.pallas_call(
        _make_t3_body(nin, nout),
        grid=(NRB,),
        in_specs=[
            *([pl.BlockSpec((RB, LANES), lambda i: (i, 0))] * nin),
            pl.BlockSpec((8, din), lambda i: (0, 0)),
            pl.BlockSpec((1, din), lambda i: (0, 0)),
            pl.BlockSpec((1, din), lambda i: (0, 0)),
            pl.BlockSpec((din, dout), lambda i: (0, 0)),
            pl.BlockSpec((NC, RB, LANES), lambda i: (0, i, 0)),
        ],
        out_specs=[pl.BlockSpec((RB, LANES), lambda i: (i, 0))] * nout,
        out_shape=[jax.ShapeDtypeStruct((N, LANES), jnp.float32)] * nout,
    )


def _make_t4_body(nblk):
    def _t4_body(*refs):
        o_refs = refs[:nblk]
        st_ref, g_ref, be_ref, bt_ref = refs[nblk:nblk + 4]
        out_ref, acc, cnt = refs[nblk + 4:]
        i = pl.program_id(0)

        @pl.when(i == 0)
        def _():
            acc[...] = jnp.zeros_like(acc)
            cnt[...] = jnp.zeros_like(cnt)

        hs = [_bn_act_block(o_refs[j][...], st_ref, g_ref, be_ref,
                            pl.ds(j * LANES, LANES))
              for j in range(nblk)]
        h = jnp.concatenate(hs, axis=1)
        onehot = (bt_ref[...] == lax.broadcasted_iota(jnp.int32, (RB, B), 1)
                  ).astype(jnp.float32)
        acc[...] += lax.dot_general(onehot, h, (((0,), (0,)), ((), ())),
                                    preferred_element_type=jnp.float32)
        cnt[...] += lax.dot_general(onehot, jnp.ones((RB, 1), jnp.float32),
                                    (((0,), (0,)), ((), ())),
                                    preferred_element_type=jnp.float32)

        @pl.when(i == NRB - 1)
        def _():
            out_ref[...] = acc[...] / jnp.maximum(cnt[...], 1.0)

    return _t4_body


def _t4_call(d):
    nblk = d // LANES
    return pl.pallas_call(
        _make_t4_body(nblk),
        grid=(NRB,),
        in_specs=[
            *([pl.BlockSpec((RB, LANES), lambda i: (i, 0))] * nblk),
            pl.BlockSpec((8, d), lambda i: (0, 0)),
            pl.BlockSpec((1, d), lambda i: (0, 0)),
            pl.BlockSpec((1, d), lambda i: (0, 0)),
            pl.BlockSpec((RB, 1), lambda i: (i, 0)),
        ],
        out_specs=pl.BlockSpec((B, d), lambda i: (0, 0)),
        out_shape=jax.ShapeDtypeStruct((B, d), jnp.float32),
        scratch_shapes=[
            pltpu.VMEM((B, d), jnp.float32),
            pltpu.VMEM((B, 1), jnp.float32),
        ],
    )


def kernel(x, edge_index, batch,
           W1, b1, g1, be1, W2, b2, g2, be2, W3, b3, g3, be3):
    row = edge_index[0].astype(jnp.int32)
    col = edge_index[1].astype(jnp.int32)
    e = row.shape[0]
    row_p = jnp.concatenate([row, jnp.zeros((EPAD - e,), jnp.int32)])
    col2 = jnp.concatenate([col, jnp.full((EPAD - e,), NPAD - 1, jnp.int32)])
    batch_c = batch.astype(jnp.int32).reshape(N, 1)

    x8 = jnp.pad(x, ((0, 0), (0, 8 - x.shape[1])))
    W18 = jnp.pad(W1, ((0, 8 - W1.shape[0]), (0, 0)))

    dp = _deg_call(col2)

    hid = W1.shape[1]
    emb = W3.shape[1]

    row2 = row_p

    y1 = _t1_call(8, hid)(x8, W18, dp)
    z1 = _spmm64(*y1, row2, col2)
    *o1, st1 = _t2_call(hid)(z1, *y1, dp, b1.reshape(1, hid))
    y2 = _t3_call(hid, hid)(*o1, st1, g1.reshape(1, hid),
                            be1.reshape(1, hid), W2, dp)
    z2 = _spmm64(*y2, row2, col2)
    *o2, st2 = _t2_call(hid)(z2, *y2, dp, b2.reshape(1, hid))
    y3 = _t3_call(hid, emb)(*o2, st2, g2.reshape(1, hid),
                            be2.reshape(1, hid), W3, dp)
    z3 = _spmm32(*y3, row2, col2)
    *o3, st3 = _t2_call(emb)(z3, *y3, dp, b3.reshape(1, emb))
    out = _t4_call(emb)(*o3, st3, g3.reshape(1, emb), be3.reshape(1, emb),
                        batch_c)
    return out


# speedup vs baseline: 1.2115x; 1.2094x over previous
import jax
import jax.numpy as jnp
from jax import lax
from jax.experimental import pallas as pl
from jax.experimental.pallas import tpu as pltpu
from jax.experimental.pallas import tpu_sc as plsc

N = 100000
B = 128
EPS = 1e-5
SLOPE = 0.01

NC = 2
NS = 16
LANES = 16

NPAD = 100352
STRIPE = NPAD // NS
ZCH = 392

GROUP = 512
SUB = 128
GSUB = GROUP // SUB

GP_SPMM = 196
EPAD = NS * GP_SPMM * GROUP
GP_DEG = EPAD // (NC * NS * GROUP)

RB = 2000
NRB = N // RB


_MESH = plsc.VectorSubcoreMesh(core_axis_name="c", subcore_axis_name="s")
_SC_PARAMS = pltpu.CompilerParams(use_tc_tiling_on_sc=False)


def _deg_body(col2, degp, acc, zb, colg, ones_v, isem, ssem):
    c = lax.axis_index("c")
    s = lax.axis_index("s")

    @pl.loop(0, STRIPE // LANES)
    def _(i):
        zb[pl.ds(i * LANES, LANES)] = jnp.zeros((LANES,), jnp.float32)

    @pl.loop(0, SUB // LANES)
    def _(i):
        ones_v[pl.ds(i * LANES, LANES)] = jnp.ones((LANES,), jnp.float32)

    pltpu.sync_copy(zb, acc.at[pl.ds(pl.multiple_of(s * STRIPE, STRIPE), STRIPE)])
    plsc.subcore_barrier()

    w = s * NC + c
    base = w * GP_DEG * GSUB
    psub = 2 * GSUB

    @pl.loop(0, GP_DEG // 2)
    def _(g):
        pltpu.sync_copy(
            col2.at[pl.ds(pl.multiple_of(base + g * psub, psub), psub)], colg)
        scps = [pltpu.async_copy(ones_v, acc.at[colg.at[u]], ssem, add=True)
                for u in range(psub)]
        for cp in scps:
            cp.wait()

    plsc.subcore_barrier()
    pltpu.sync_copy(
        acc.at[pl.ds(pl.multiple_of(s * STRIPE, STRIPE), STRIPE)],
        degp.at[pl.ds(pl.multiple_of(c * NPAD + s * STRIPE, STRIPE), STRIPE)])


_deg_call = pl.kernel(
    _deg_body,
    out_type=jax.ShapeDtypeStruct((NC * NPAD,), jnp.float32),
    mesh=_MESH,
    compiler_params=_SC_PARAMS,
    scratch_types=[
        pltpu.VMEM_SHARED((NPAD,), jnp.float32),
        pltpu.VMEM((STRIPE,), jnp.float32),
        pltpu.VMEM((2 * GSUB, SUB), jnp.int32),
        pltpu.VMEM((SUB,), jnp.float32),
        pltpu.SemaphoreType.DMA,
        pltpu.SemaphoreType.DMA,
    ],
)


def _make_spmm(nblk):
    bpc = nblk // NC
    pair = 2 * GROUP
    psub = 2 * GSUB

    def body(*refs):
        yf = refs[0]
        rowbs = refs[1:1 + nblk]
        col2, z = refs[1 + nblk:3 + nblk]
        (acc, zb, rowp, colp, gbuf0, gbuf1,
         isem, gsem0, gsem1, ssem0, ssem1, wsem) = refs[3 + nblk:]
        c = lax.axis_index("c")
        s = lax.axis_index("s")

        @pl.loop(0, ZCH)
        def _(i):
            zb[i] = jnp.zeros((LANES,), jnp.float32)

        for core in range(NC):
            @pl.when(c == core)
            def _():
                for bi in range(bpc):
                    blk = bi * NC + core
                    rtab = rowbs[blk]

                    zcps = [
                        pltpu.async_copy(
                            zb,
                            acc.at[pl.ds(
                                pl.multiple_of(s * STRIPE + k * ZCH, ZCH),
                                ZCH)],
                            wsem)
                        for k in range(STRIPE // ZCH)
                    ]
                    for cp in zcps:
                        cp.wait()
                    plsc.subcore_barrier()

                    @pl.loop(0, GP_SPMM // 2)
                    def _(i):
                        eb = pl.multiple_of(
                            s * GP_SPMM * GROUP + i * pair, pair)
                        eb128 = pl.multiple_of(
                            s * GP_SPMM * GSUB + i * psub, psub)
                        icps = [pltpu.async_copy(rtab.at[pl.ds(eb, pair)],
                                                 rowp, isem),
                                pltpu.async_copy(col2.at[pl.ds(eb128, psub)],
                                                 colp, isem)]
                        for cp in icps:
                            cp.wait()

                        gcps0 = [pltpu.async_copy(
                            yf.at[rowp.at[pl.ds(u * SUB, SUB)]],
                            gbuf0.at[pl.ds(u * SUB, SUB)], gsem0)
                            for u in range(GSUB)]
                        gcps1 = [pltpu.async_copy(
                            yf.at[rowp.at[pl.ds(GROUP + u * SUB, SUB)]],
                            gbuf1.at[pl.ds(u * SUB, SUB)], gsem1)
                            for u in range(GSUB)]
                        for cp in gcps0:
                            cp.wait()
                        scps0 = [pltpu.async_copy(
                            gbuf0.at[pl.ds(u * SUB, SUB)],
                            acc.at[colp.at[u]], ssem0, add=True)
                            for u in range(GSUB)]
                        for cp in gcps1:
                            cp.wait()
                        scps1 = [pltpu.async_copy(
                            gbuf1.at[pl.ds(u * SUB, SUB)],
                            acc.at[colp.at[GSUB + u]], ssem1, add=True)
                            for u in range(GSUB)]
                        for cp in scps0:
                            cp.wait()
                        for cp in scps1:
                            cp.wait()

                    plsc.subcore_barrier()

                    r = pl.ds(pl.multiple_of(s * STRIPE, STRIPE), STRIPE)
                    pltpu.sync_copy(acc.at[r], z.at[blk, r])
                    plsc.subcore_barrier()

    return pl.kernel(
        body,
        out_type=jax.ShapeDtypeStruct((nblk, NPAD, LANES), jnp.float32),
        mesh=_MESH,
        compiler_params=_SC_PARAMS,
        scratch_types=[
            pltpu.VMEM_SHARED((NPAD, LANES), jnp.float32),
            pltpu.VMEM((ZCH, LANES), jnp.float32),
            pltpu.VMEM((pair,), jnp.int32),
            pltpu.VMEM((psub, SUB), jnp.int32),
            pltpu.VMEM((GROUP, LANES), jnp.float32),
            pltpu.VMEM((GROUP, LANES), jnp.float32),
            pltpu.SemaphoreType.DMA,
            pltpu.SemaphoreType.DMA,
            pltpu.SemaphoreType.DMA,
            pltpu.SemaphoreType.DMA,
            pltpu.SemaphoreType.DMA,
            pltpu.SemaphoreType.DMA,
        ],
    )


_spmm64 = _make_spmm(4)
_spmm32 = _make_spmm(2)


def _t1_body(x_ref, w_ref, d0_ref, d1_ref, y_ref):
    dinv = lax.rsqrt(d0_ref[...] + d1_ref[...] + 1.0)
    y_ref[...] = jnp.dot(x_ref[...], w_ref[...],
                         preferred_element_type=jnp.float32) * dinv


def _t1_call(din, dout):
    return pl.pallas_call(
        _t1_body,
        grid=(NRB,),
        in_specs=[
            pl.BlockSpec((RB, din), lambda i: (i, 0)),
            pl.BlockSpec((din, dout), lambda i: (0, 0)),
            pl.BlockSpec((RB, 1), lambda i: (i, 0)),
            pl.BlockSpec((RB, 1), lambda i: (i, 0)),
        ],
        out_specs=pl.BlockSpec((RB, dout), lambda i: (i, 0)),
        out_shape=jax.ShapeDtypeStruct((N, dout), jnp.float32),
    )


def _make_t2_body(nblk):
    def _t2_body(z_ref, y_ref, d0_ref, d1_ref, b_ref, o_ref, st_ref):
        i = pl.program_id(0)
        dinv = lax.rsqrt(d0_ref[...] + d1_ref[...] + 1.0)
        zcat = jnp.concatenate([z_ref[j] for j in range(nblk)], axis=1)
        o = (zcat + y_ref[...]) * dinv + b_ref[...]
        o_ref[...] = o

        @pl.when(i == 0)
        def _():
            st_ref[...] = jnp.zeros_like(st_ref)

        st_ref[0:1, :] += jnp.sum(o, axis=0, keepdims=True)
        st_ref[1:2, :] += jnp.sum(o * o, axis=0, keepdims=True)

    return _t2_body


def _t2_call(d):
    nblk = d // LANES
    return pl.pallas_call(
        _make_t2_body(nblk),
        grid=(NRB,),
        in_specs=[
            pl.BlockSpec((nblk, RB, LANES), lambda i: (0, i, 0)),
            pl.BlockSpec((RB, d), lambda i: (i, 0)),
            pl.BlockSpec((RB, 1), lambda i: (i, 0)),
            pl.BlockSpec((RB, 1), lambda i: (i, 0)),
            pl.BlockSpec((1, d), lambda i: (0, 0)),
        ],
        out_specs=[
            pl.BlockSpec((RB, d), lambda i: (i, 0)),
            pl.BlockSpec((8, d), lambda i: (0, 0)),
        ],
        out_shape=[
            jax.ShapeDtypeStruct((N, d), jnp.float32),
            jax.ShapeDtypeStruct((8, d), jnp.float32),
        ],
    )


def _bn_act(o_ref, st_ref, g_ref, be_ref):
    mean = st_ref[0:1, :] * (1.0 / N)
    var = st_ref[1:2, :] * (1.0 / N) - mean * mean
    xn = (o_ref[...] - mean) * lax.rsqrt(var + EPS) * g_ref[...] + be_ref[...]
    return jnp.where(xn >= 0, xn, SLOPE * xn)


def _t3_body(o_ref, st_ref, g_ref, be_ref, w_ref, d0_ref, d1_ref, y_ref):
    h = _bn_act(o_ref, st_ref, g_ref, be_ref)
    dinv = lax.rsqrt(d0_ref[...] + d1_ref[...] + 1.0)
    y_ref[...] = jnp.dot(h, w_ref[...],
                         preferred_element_type=jnp.float32) * dinv


def _t3_call(din, dout):
    return pl.pallas_call(
        _t3_body,
        grid=(NRB,),
        in_specs=[
            pl.BlockSpec((RB, din), lambda i: (i, 0)),
            pl.BlockSpec((8, din), lambda i: (0, 0)),
            pl.BlockSpec((1, din), lambda i: (0, 0)),
            pl.BlockSpec((1, din), lambda i: (0, 0)),
            pl.BlockSpec((din, dout), lambda i: (0, 0)),
            pl.BlockSpec((RB, 1), lambda i: (i, 0)),
            pl.BlockSpec((RB, 1), lambda i: (i, 0)),
        ],
        out_specs=pl.BlockSpec((RB, dout), lambda i: (i, 0)),
        out_shape=jax.ShapeDtypeStruct((N, dout), jnp.float32),
    )


def _t4_body(o_ref, st_ref, g_ref, be_ref, bt_ref, out_ref, acc, cnt):
    i = pl.program_id(0)

    @pl.when(i == 0)
    def _():
        acc[...] = jnp.zeros_like(acc)
        cnt[...] = jnp.zeros_like(cnt)

    h = _bn_act(o_ref, st_ref, g_ref, be_ref)
    onehot = (bt_ref[...] == lax.broadcasted_iota(jnp.int32, (RB, B), 1)
              ).astype(jnp.float32)
    acc[...] += lax.dot_general(onehot, h, (((0,), (0,)), ((), ())),
                                preferred_element_type=jnp.float32)
    cnt[...] += lax.dot_general(onehot, jnp.ones((RB, 1), jnp.float32),
                                (((0,), (0,)), ((), ())),
                                preferred_element_type=jnp.float32)

    @pl.when(i == NRB - 1)
    def _():
        out_ref[...] = acc[...] / jnp.maximum(cnt[...], 1.0)


def _t4_call(d):
    return pl.pallas_call(
        _t4_body,
        grid=(NRB,),
        in_specs=[
            pl.BlockSpec((RB, d), lambda i: (i, 0)),
            pl.BlockSpec((8, d), lambda i: (0, 0)),
            pl.BlockSpec((1, d), lambda i: (0, 0)),
            pl.BlockSpec((1, d), lambda i: (0, 0)),
            pl.BlockSpec((RB, 1), lambda i: (i, 0)),
        ],
        out_specs=pl.BlockSpec((B, d), lambda i: (0, 0)),
        out_shape=jax.ShapeDtypeStruct((B, d), jnp.float32),
        scratch_shapes=[
            pltpu.VMEM((B, d), jnp.float32),
            pltpu.VMEM((B, 1), jnp.float32),
        ],
    )


def kernel(x, edge_index, batch,
           W1, b1, g1, be1, W2, b2, g2, be2, W3, b3, g3, be3):
    row = edge_index[0].astype(jnp.int32)
    col = edge_index[1].astype(jnp.int32)
    e = row.shape[0]
    row_p = jnp.concatenate([row, jnp.zeros((EPAD - e,), jnp.int32)])
    col_p = jnp.concatenate([col, jnp.full((EPAD - e,), NPAD - 1, jnp.int32)])
    col2 = col_p.reshape(EPAD // SUB, SUB)
    batch_c = batch.astype(jnp.int32).reshape(N, 1)

    x8 = jnp.pad(x, ((0, 0), (0, 8 - x.shape[1])))
    W18 = jnp.pad(W1, ((0, 8 - W1.shape[0]), (0, 0)))

    degp = _deg_call(col2).reshape(NC, NPAD, 1)
    d0 = degp[0]
    d1 = degp[1]

    hid = W1.shape[1]
    emb = W3.shape[1]

    y1 = _t1_call(8, hid)(x8, W18, d0, d1)
    rb4 = tuple(row_p * 4 + b for b in range(4))
    rb2 = tuple(row_p * 2 + b for b in range(2))

    z1 = _spmm64(y1.reshape(N * 4, LANES), *rb4, col2)
    o1, st1 = _t2_call(hid)(z1, y1, d0, d1, b1.reshape(1, hid))
    y2 = _t3_call(hid, hid)(o1, st1, g1.reshape(1, hid), be1.reshape(1, hid),
                            W2, d0, d1)
    z2 = _spmm64(y2.reshape(N * 4, LANES), *rb4, col2)
    o2, st2 = _t2_call(hid)(z2, y2, d0, d1, b2.reshape(1, hid))
    y3 = _t3_call(hid, emb)(o2, st2, g2.reshape(1, hid), be2.reshape(1, hid),
                            W3, d0, d1)
    z3 = _spmm32(y3.reshape(N * 2, LANES), *rb2, col2)
    o3, st3 = _t2_call(emb)(z3, y3, d0, d1, b3.reshape(1, emb))
    out = _t4_call(emb)(o3, st3, g3.reshape(1, emb), be3.reshape(1, emb),
                        batch_c)
    return out
